# Initial kernel scaffold; baseline (speedup 1.0000x reference)
import jax
import jax.numpy as jnp
from jax.experimental import pallas as pl


def kernel(x, edge_index, edge_attr, params):
    raise NotImplementedError("write your pallas kernel here")


# speedup vs baseline: 2.0533x; 2.0533x over previous
import functools

import jax
import jax.numpy as jnp
from jax import lax
from jax.experimental import pallas as pl
from jax.experimental.pallas import tpu as pltpu
from jax.experimental.pallas import tpu_sc as plsc

F32 = jnp.float32
LN_EPS = 1e-5


def _ln(y, g, b):
    mu = jnp.mean(y, axis=-1, keepdims=True)
    var = jnp.mean((y - mu) ** 2, axis=-1, keepdims=True)
    return (y - mu) * lax.rsqrt(var + LN_EPS) * g + b


def _mlp_ln_body(x_ref, w1_ref, b1_ref, w2_ref, b2_ref, g_ref, be_ref, o_ref):
    h = jnp.maximum(
        jnp.dot(x_ref[...], w1_ref[...], preferred_element_type=F32) + b1_ref[...], 0.0)
    y = jnp.dot(h, w2_ref[...], preferred_element_type=F32) + b2_ref[...]
    o_ref[...] = _ln(y, g_ref[...], be_ref[...])


def _edge_body(e_ref, gs_ref, gd_ref, a_ref, b_ref, c_ref, b1_ref, w2_ref,
               b2_ref, lg_ref, lb_ref, o_ref):
    pre = (jnp.dot(e_ref[...], a_ref[...], preferred_element_type=F32)
           + jnp.dot(gs_ref[...], b_ref[...], preferred_element_type=F32)
           + jnp.dot(gd_ref[...], c_ref[...], preferred_element_type=F32)
           + b1_ref[...])
    h = jnp.maximum(pre, 0.0)
    y = jnp.dot(h, w2_ref[...], preferred_element_type=F32) + b2_ref[...]
    o_ref[...] = e_ref[...] + _ln(y, lg_ref[...], lb_ref[...])


def _node_body(x_ref, agg_ref, p_ref, q_ref, b1_ref, w2_ref, b2_ref, lg_ref,
               lb_ref, o_ref):
    agg = agg_ref[0] + agg_ref[1]
    pre = (jnp.dot(x_ref[...], p_ref[...], preferred_element_type=F32)
           + jnp.dot(agg, q_ref[...], preferred_element_type=F32)
           + b1_ref[...])
    h = jnp.maximum(pre, 0.0)
    y = jnp.dot(h, w2_ref[...], preferred_element_type=F32) + b2_ref[...]
    o_ref[...] = x_ref[...] + _ln(y, lg_ref[...], lb_ref[...])


def _dec_body(x_ref, w1_ref, b1_ref, w2_ref, b2_ref, o_ref):
    h = jnp.maximum(
        jnp.dot(x_ref[...], w1_ref[...], preferred_element_type=F32) + b1_ref[...], 0.0)
    o_ref[...] = jnp.dot(h, w2_ref[...], preferred_element_type=F32) + b2_ref[...]


def _full(shape):
    return pl.BlockSpec(shape, lambda i: (0,) * len(shape))


def _rows(bs, shape_rest):
    return pl.BlockSpec((bs,) + shape_rest, lambda i: (i,) + (0,) * len(shape_rest))


def _mlp_ln_call(x, w1, b1, w2, b2, g, be, block):
    n, fin = x.shape
    fout = w2.shape[1]
    grid = (n // block,)
    return pl.pallas_call(
        _mlp_ln_body,
        grid=grid,
        in_specs=[_rows(block, (fin,)), _full(w1.shape), _full(b1.shape),
                  _full(w2.shape), _full(b2.shape), _full(g.shape), _full(be.shape)],
        out_specs=_rows(block, (fout,)),
        out_shape=jax.ShapeDtypeStruct((n, fout), F32),
    )(x, w1, b1, w2, b2, g, be)


_CH = 128


@functools.lru_cache(maxsize=None)
def _make_sc_gather(n_nodes, n_edges, feat):
    info = plsc.get_sparse_core_info()
    nw = info.num_cores * info.num_subcores
    per_w = n_edges // nw
    assert n_edges % nw == 0
    n_full = per_w // _CH
    rem = per_w - n_full * _CH
    mesh = plsc.VectorSubcoreMesh(core_axis_name="c", subcore_axis_name="s")

    scratch = [
        pltpu.VMEM((_CH,), jnp.int32), pltpu.VMEM((_CH,), jnp.int32),
        pltpu.VMEM((_CH, feat), F32), pltpu.VMEM((_CH, feat), F32),
        pltpu.SemaphoreType.DMA, pltpu.SemaphoreType.DMA,
    ]
    if rem:
        scratch += [
            pltpu.VMEM((rem,), jnp.int32), pltpu.VMEM((rem,), jnp.int32),
            pltpu.VMEM((rem, feat), F32), pltpu.VMEM((rem, feat), F32),
        ]

    @functools.partial(
        pl.kernel,
        out_type=(jax.ShapeDtypeStruct((n_edges, feat), F32),
                  jax.ShapeDtypeStruct((n_edges, feat), F32)),
        mesh=mesh,
        scratch_types=scratch,
        compiler_params=pltpu.CompilerParams(use_tc_tiling_on_sc=False),
    )
    def gather(x_hbm, src_hbm, dst_hbm, gs_hbm, gd_hbm, *refs):
        if rem:
            (sidx, didx, rows_a, rows_b, sem_a, sem_b,
             sidx_t, didx_t, rows_at, rows_bt) = refs
        else:
            sidx, didx, rows_a, rows_b, sem_a, sem_b = refs
        wid = lax.axis_index("s") * info.num_cores + lax.axis_index("c")
        base0 = wid * per_w

        def chunk(base, si, di, ra, rb, k):
            pltpu.sync_copy(src_hbm.at[pl.ds(base, k)], si)
            pltpu.sync_copy(dst_hbm.at[pl.ds(base, k)], di)
            ca = pltpu.async_copy(x_hbm.at[si], ra, sem_a)
            cb = pltpu.async_copy(x_hbm.at[di], rb, sem_b)
            ca.wait()
            cb.wait()
            pltpu.sync_copy(ra, gs_hbm.at[pl.ds(base, k)])
            pltpu.sync_copy(rb, gd_hbm.at[pl.ds(base, k)])

        def body(j, carry):
            chunk(base0 + j * _CH, sidx, didx, rows_a, rows_b, _CH)
            return carry

        lax.fori_loop(0, n_full, body, 0)
        if rem:
            chunk(base0 + n_full * _CH, sidx_t, didx_t, rows_at, rows_bt, rem)

    return gather


@functools.lru_cache(maxsize=None)
def _make_sc_scatter(n_nodes_pad, n_edges, feat):
    info = plsc.get_sparse_core_info()
    nc, ns = info.num_cores, info.num_subcores
    nw = nc * ns
    per_w = n_edges // nw
    assert n_edges % nw == 0
    n_full = per_w // _CH
    rem = per_w - n_full * _CH
    rows_per_s = n_nodes_pad // ns
    assert n_nodes_pad % (ns * _CH) == 0
    zgroups = rows_per_s // _CH
    mesh = plsc.VectorSubcoreMesh(core_axis_name="c", subcore_axis_name="s")

    scratch = [
        pltpu.VMEM((_CH,), jnp.int32),
        pltpu.VMEM((_CH, feat), F32),
        pltpu.VMEM_SHARED((n_nodes_pad, feat), F32),
    ]
    if rem:
        scratch += [pltpu.VMEM((rem,), jnp.int32), pltpu.VMEM((rem, feat), F32)]

    @functools.partial(
        pl.kernel,
        out_type=jax.ShapeDtypeStruct((nc, n_nodes_pad, feat), F32),
        mesh=mesh,
        scratch_types=scratch,
        compiler_params=pltpu.CompilerParams(use_tc_tiling_on_sc=False),
    )
    def scatter(e_hbm, dst_hbm, out_hbm, *refs):
        if rem:
            didx, rows, acc, didx_t, rows_t = refs
        else:
            didx, rows, acc = refs
        cid = lax.axis_index("c")
        sid = lax.axis_index("s")
        wid = sid * nc + cid
        base0 = wid * per_w

        def zrow(i, carry):
            for c in range(feat // 16):
                rows[i, pl.ds(c * 16, 16)] = jnp.zeros((16,), F32)
            return carry

        lax.fori_loop(0, _CH, zrow, 0)
        for t in range(zgroups):
            pltpu.sync_copy(rows, acc.at[pl.ds(sid * rows_per_s + t * _CH, _CH)])
        plsc.subcore_barrier()

        def chunk(base, di, rw, k):
            pltpu.sync_copy(dst_hbm.at[pl.ds(base, k)], di)
            pltpu.sync_copy(e_hbm.at[pl.ds(base, k)], rw)
            pltpu.sync_copy(rw, acc.at[di], add=True)

        def body(j, carry):
            chunk(base0 + j * _CH, didx, rows, _CH)
            return carry

        lax.fori_loop(0, n_full, body, 0)
        if rem:
            chunk(base0 + n_full * _CH, didx_t, rows_t, rem)
        plsc.subcore_barrier()

        pltpu.sync_copy(acc.at[pl.ds(sid * rows_per_s, rows_per_s)],
                        out_hbm.at[cid, pl.ds(sid * rows_per_s, rows_per_s)])

    return scatter


def _row2(v):
    return v.reshape(1, -1)


def kernel(x, edge_index, edge_attr, params):
    n, node_f = x.shape
    e_cnt = edge_index.shape[1]
    latent = params['enc_node']['ln'][0].shape[0]
    n_pad = 10240
    bn = 2000
    be = 3200

    src = edge_index[0]
    dst = edge_index[1]

    (w1n, b1n), (w2n, b2n) = params['enc_node']['mlp']
    gn, ben = params['enc_node']['ln']
    xh = _mlp_ln_call(x, w1n, _row2(b1n), w2n, _row2(b2n), _row2(gn), _row2(ben), bn)

    (w1e, b1e), (w2e, b2e) = params['enc_edge']['mlp']
    ge, bee = params['enc_edge']['ln']
    eh = _mlp_ln_call(edge_attr, w1e, _row2(b1e), w2e, _row2(b2e), _row2(ge),
                      _row2(bee), be)

    sc_gather = _make_sc_gather(n, e_cnt, latent)
    sc_scatter = _make_sc_scatter(n_pad, e_cnt, latent)

    edge_grid = (e_cnt // be,)
    node_grid = (n // bn,)

    edge_call = pl.pallas_call(
        _edge_body,
        grid=edge_grid,
        in_specs=[_rows(be, (latent,)), _rows(be, (latent,)), _rows(be, (latent,)),
                  _full((latent, latent)), _full((latent, latent)),
                  _full((latent, latent)), _full((1, latent)),
                  _full((latent, latent)), _full((1, latent)),
                  _full((1, latent)), _full((1, latent))],
        out_specs=_rows(be, (latent,)),
        out_shape=jax.ShapeDtypeStruct((e_cnt, latent), F32),
    )

    node_call = pl.pallas_call(
        _node_body,
        grid=node_grid,
        in_specs=[_rows(bn, (latent,)),
                  pl.BlockSpec((2, bn, latent), lambda i: (0, i, 0)),
                  _full((latent, latent)), _full((latent, latent)),
                  _full((1, latent)), _full((latent, latent)), _full((1, latent)),
                  _full((1, latent)), _full((1, latent))],
        out_specs=_rows(bn, (latent,)),
        out_shape=jax.ShapeDtypeStruct((n, latent), F32),
    )

    for p in params['proc']:
        (ew1, eb1), (ew2, eb2) = p['edge_mlp']
        elg, elb = p['edge_ln']
        (nw1, nb1), (nw2, nb2) = p['node_mlp']
        nlg, nlb = p['node_ln']
        a_m = ew1[:latent]
        b_m = ew1[latent:2 * latent]
        c_m = ew1[2 * latent:]
        p_m = nw1[:latent]
        q_m = nw1[latent:]

        gs, gd = sc_gather(xh, src, dst)
        eh = edge_call(eh, gs, gd, a_m, b_m, c_m, _row2(eb1), ew2, _row2(eb2),
                       _row2(elg), _row2(elb))
        agg = sc_scatter(eh, dst)
        xh = node_call(xh, agg, p_m, q_m, _row2(nb1), nw2, _row2(nb2),
                       _row2(nlg), _row2(nlb))

    (dw1, db1), (dw2, db2) = params['dec']
    out_dim = dw2.shape[1]
    pad = 8 - out_dim
    dw2p = jnp.pad(dw2, ((0, 0), (0, pad)))
    db2p = jnp.pad(db2, ((0, pad),))
    dec = pl.pallas_call(
        _dec_body,
        grid=node_grid,
        in_specs=[_rows(bn, (latent,)), _full((latent, latent)), _full((1, latent)),
                  _full((latent, 8)), _full((1, 8))],
        out_specs=_rows(bn, (8,)),
        out_shape=jax.ShapeDtypeStruct((n, 8), F32),
    )(xh, dw1, _row2(db1), dw2p, _row2(db2p))
    return dec[:, :out_dim]


# speedup vs baseline: 2.4987x; 1.2169x over previous
import functools

import jax
import jax.numpy as jnp
from jax import lax
from jax.experimental import pallas as pl
from jax.experimental.pallas import tpu as pltpu
from jax.experimental.pallas import tpu_sc as plsc

F32 = jnp.float32
LN_EPS = 1e-5


def _ln(y, g, b):
    mu = jnp.mean(y, axis=-1, keepdims=True)
    var = jnp.mean((y - mu) ** 2, axis=-1, keepdims=True)
    return (y - mu) * lax.rsqrt(var + LN_EPS) * g + b


def _mlp_ln_body(x_ref, w1_ref, b1_ref, w2_ref, b2_ref, g_ref, be_ref, o_ref):
    h = jnp.maximum(
        jnp.dot(x_ref[...], w1_ref[...], preferred_element_type=F32) + b1_ref[...], 0.0)
    y = jnp.dot(h, w2_ref[...], preferred_element_type=F32) + b2_ref[...]
    o_ref[...] = _ln(y, g_ref[...], be_ref[...])


def _edge_body(e_ref, gs_ref, gd_ref, a_ref, b_ref, c_ref, b1_ref, w2_ref,
               b2_ref, lg_ref, lb_ref, o_ref):
    pre = (jnp.dot(e_ref[...], a_ref[...], preferred_element_type=F32)
           + jnp.dot(gs_ref[...], b_ref[...], preferred_element_type=F32)
           + jnp.dot(gd_ref[...], c_ref[...], preferred_element_type=F32)
           + b1_ref[...])
    h = jnp.maximum(pre, 0.0)
    y = jnp.dot(h, w2_ref[...], preferred_element_type=F32) + b2_ref[...]
    o_ref[...] = e_ref[...] + _ln(y, lg_ref[...], lb_ref[...])


def _node_body(x_ref, agg_ref, p_ref, q_ref, b1_ref, w2_ref, b2_ref, lg_ref,
               lb_ref, o_ref):
    agg = agg_ref[0] + agg_ref[1]
    pre = (jnp.dot(x_ref[...], p_ref[...], preferred_element_type=F32)
           + jnp.dot(agg, q_ref[...], preferred_element_type=F32)
           + b1_ref[...])
    h = jnp.maximum(pre, 0.0)
    y = jnp.dot(h, w2_ref[...], preferred_element_type=F32) + b2_ref[...]
    o_ref[...] = x_ref[...] + _ln(y, lg_ref[...], lb_ref[...])


def _dec_body(x_ref, w1_ref, b1_ref, w2_ref, b2_ref, o_ref):
    h = jnp.maximum(
        jnp.dot(x_ref[...], w1_ref[...], preferred_element_type=F32) + b1_ref[...], 0.0)
    o_ref[...] = jnp.dot(h, w2_ref[...], preferred_element_type=F32) + b2_ref[...]


def _full(shape):
    return pl.BlockSpec(shape, lambda i: (0,) * len(shape))


def _rows(bs, shape_rest):
    return pl.BlockSpec((bs,) + shape_rest, lambda i: (i,) + (0,) * len(shape_rest))


def _mlp_ln_call(x, w1, b1, w2, b2, g, be, block):
    n, fin = x.shape
    fout = w2.shape[1]
    grid = (n // block,)
    return pl.pallas_call(
        _mlp_ln_body,
        grid=grid,
        in_specs=[_rows(block, (fin,)), _full(w1.shape), _full(b1.shape),
                  _full(w2.shape), _full(b2.shape), _full(g.shape), _full(be.shape)],
        out_specs=_rows(block, (fout,)),
        out_shape=jax.ShapeDtypeStruct((n, fout), F32),
    )(x, w1, b1, w2, b2, g, be)


_CH = 128


@functools.lru_cache(maxsize=None)
def _make_sc_gather(n_nodes, n_edges, feat):
    info = plsc.get_sparse_core_info()
    nw = info.num_cores * info.num_subcores
    per_w = n_edges // nw
    assert n_edges % nw == 0
    n_full = per_w // _CH
    rem = per_w - n_full * _CH
    assert n_full >= 4 and n_full % 2 == 0
    mesh = plsc.VectorSubcoreMesh(core_axis_name="c", subcore_axis_name="s")

    scratch = [
        pltpu.VMEM((2, _CH), jnp.int32), pltpu.VMEM((2, _CH), jnp.int32),
        pltpu.VMEM((2, _CH, feat), F32), pltpu.VMEM((2, _CH, feat), F32),
    ] + [pltpu.SemaphoreType.DMA] * 12
    if rem:
        scratch += [
            pltpu.VMEM((rem,), jnp.int32), pltpu.VMEM((rem,), jnp.int32),
            pltpu.VMEM((rem, feat), F32), pltpu.VMEM((rem, feat), F32),
        ]

    @functools.partial(
        pl.kernel,
        out_type=(jax.ShapeDtypeStruct((n_edges, feat), F32),
                  jax.ShapeDtypeStruct((n_edges, feat), F32)),
        mesh=mesh,
        scratch_types=scratch,
        compiler_params=pltpu.CompilerParams(use_tc_tiling_on_sc=False),
    )
    def gather(x_hbm, src_hbm, dst_hbm, gs_hbm, gd_hbm, *refs):
        sidx, didx, ra, rb = refs[0:4]
        semis = refs[4:6]
        semid = refs[6:8]
        semga = refs[8:10]
        semgb = refs[10:12]
        semwa = refs[12:14]
        semwb = refs[14:16]
        if rem:
            sidx_t, didx_t, rows_at, rows_bt = refs[16:20]
        wid = lax.axis_index("s") * info.num_cores + lax.axis_index("c")
        base0 = wid * per_w

        def i_issue(c, s):
            pltpu.async_copy(src_hbm.at[pl.ds(base0 + c * _CH, _CH)],
                             sidx.at[s], semis[s])
            pltpu.async_copy(dst_hbm.at[pl.ds(base0 + c * _CH, _CH)],
                             didx.at[s], semid[s])

        def i_wait(s):
            pltpu.make_async_copy(src_hbm.at[pl.ds(0, _CH)], sidx.at[s],
                                  semis[s]).wait()
            pltpu.make_async_copy(dst_hbm.at[pl.ds(0, _CH)], didx.at[s],
                                  semid[s]).wait()

        def g_issue(s):
            pltpu.async_copy(x_hbm.at[sidx.at[s]], ra.at[s], semga[s])
            pltpu.async_copy(x_hbm.at[didx.at[s]], rb.at[s], semgb[s])

        def g_wait(s):
            pltpu.make_async_copy(gs_hbm.at[pl.ds(0, _CH)], ra.at[s],
                                  semga[s]).wait()
            pltpu.make_async_copy(gs_hbm.at[pl.ds(0, _CH)], rb.at[s],
                                  semgb[s]).wait()

        def w_issue(c, s):
            pltpu.async_copy(ra.at[s], gs_hbm.at[pl.ds(base0 + c * _CH, _CH)],
                             semwa[s])
            pltpu.async_copy(rb.at[s], gd_hbm.at[pl.ds(base0 + c * _CH, _CH)],
                             semwb[s])

        def w_wait(s):
            pltpu.make_async_copy(ra.at[s], gs_hbm.at[pl.ds(0, _CH)],
                                  semwa[s]).wait()
            pltpu.make_async_copy(rb.at[s], gd_hbm.at[pl.ds(0, _CH)],
                                  semwb[s]).wait()

        def iteration(j, s, do_next_gather, do_write_wait, do_idx_prefetch):
            s1 = 1 - s
            if do_next_gather:
                i_wait(s1)
                if do_write_wait:
                    w_wait(s1)
                g_issue(s1)
            g_wait(s)
            w_issue(j, s)
            if do_idx_prefetch:
                i_issue(j + 2, s)

        i_issue(0, 0)
        i_issue(1, 1)
        i_wait(0)
        g_issue(0)
        iteration(0, 0, True, False, True)
        def body(g, carry):
            j = 2 * g + 1
            iteration(j, 1, True, True, True)
            iteration(j + 1, 0, True, True, True)
            return carry

        lax.fori_loop(0, n_full // 2 - 2, body, 0)
        iteration(n_full - 3, 1, True, True, True)
        iteration(n_full - 2, 0, True, True, False)
        iteration(n_full - 1, 1, False, False, False)
        w_wait(0)
        w_wait(1)

        if rem:
            base = base0 + n_full * _CH
            pltpu.sync_copy(src_hbm.at[pl.ds(base, rem)], sidx_t)
            pltpu.sync_copy(dst_hbm.at[pl.ds(base, rem)], didx_t)
            ca = pltpu.async_copy(x_hbm.at[sidx_t], rows_at, semga[0])
            cb = pltpu.async_copy(x_hbm.at[didx_t], rows_bt, semgb[0])
            ca.wait()
            cb.wait()
            pltpu.sync_copy(rows_at, gs_hbm.at[pl.ds(base, rem)])
            pltpu.sync_copy(rows_bt, gd_hbm.at[pl.ds(base, rem)])

    return gather


@functools.lru_cache(maxsize=None)
def _make_sc_scatter(n_nodes_pad, n_edges, feat):
    info = plsc.get_sparse_core_info()
    nc, ns = info.num_cores, info.num_subcores
    nw = nc * ns
    per_w = n_edges // nw
    assert n_edges % nw == 0
    n_full = per_w // _CH
    rem = per_w - n_full * _CH
    rows_per_s = n_nodes_pad // ns
    assert n_nodes_pad % (ns * _CH) == 0
    zgroups = rows_per_s // _CH
    mesh = plsc.VectorSubcoreMesh(core_axis_name="c", subcore_axis_name="s")

    assert n_full >= 6 and (n_full - 3) % 3 == 0
    scratch = [
        pltpu.VMEM((3, _CH), jnp.int32),
        pltpu.VMEM((3, _CH, feat), F32),
        pltpu.VMEM_SHARED((n_nodes_pad, feat), F32),
    ] + [pltpu.SemaphoreType.DMA] * 9
    if rem:
        scratch += [pltpu.VMEM((rem,), jnp.int32), pltpu.VMEM((rem, feat), F32)]

    @functools.partial(
        pl.kernel,
        out_type=jax.ShapeDtypeStruct((nc, n_nodes_pad, feat), F32),
        mesh=mesh,
        scratch_types=scratch,
        compiler_params=pltpu.CompilerParams(use_tc_tiling_on_sc=False),
    )
    def scatter(e_hbm, dst_hbm, out_hbm, *refs):
        didx, rows, acc = refs[0:3]
        semli = refs[3:6]
        semlr = refs[6:9]
        semsc = refs[9:12]
        if rem:
            didx_t, rows_t = refs[12:14]
        cid = lax.axis_index("c")
        sid = lax.axis_index("s")
        wid = sid * nc + cid
        base0 = wid * per_w

        def zrow(i, carry):
            for c in range(feat // 16):
                rows[0, i, pl.ds(c * 16, 16)] = jnp.zeros((16,), F32)
            return carry

        lax.fori_loop(0, _CH, zrow, 0)
        for t in range(zgroups):
            pltpu.sync_copy(rows.at[0],
                            acc.at[pl.ds(sid * rows_per_s + t * _CH, _CH)])
        plsc.subcore_barrier()

        def l_issue(c, s):
            pltpu.async_copy(dst_hbm.at[pl.ds(base0 + c * _CH, _CH)],
                             didx.at[s], semli[s])
            pltpu.async_copy(e_hbm.at[pl.ds(base0 + c * _CH, _CH)],
                             rows.at[s], semlr[s])

        def l_wait(s):
            pltpu.make_async_copy(dst_hbm.at[pl.ds(0, _CH)], didx.at[s],
                                  semli[s]).wait()
            pltpu.make_async_copy(e_hbm.at[pl.ds(0, _CH)], rows.at[s],
                                  semlr[s]).wait()

        def a_issue(s):
            pltpu.async_copy(rows.at[s], acc.at[didx.at[s]], semsc[s],
                             add=True)

        def a_wait(s):
            pltpu.make_async_copy(rows.at[s], acc.at[didx.at[s]],
                                  semsc[s]).wait()

        def iteration(j, s, do_scatter_wait, do_prefetch):
            l_wait(s)
            a_issue(s)
            if do_scatter_wait:
                a_wait((s + 2) % 3)
            if do_prefetch:
                l_issue(j + 2, (s + 2) % 3)

        l_issue(0, 0)
        l_issue(1, 1)
        l_issue(2, 2)
        iteration(0, 0, False, False)
        def body(g, carry):
            j = 3 * g + 1
            iteration(j, 1, True, True)
            iteration(j + 1, 2, True, True)
            iteration(j + 2, 0, True, True)
            return carry

        lax.fori_loop(0, (n_full - 3) // 3, body, 0)
        j0 = n_full - 2
        iteration(j0, j0 % 3, True, False)
        iteration(j0 + 1, (j0 + 1) % 3, True, False)
        a_wait((j0 + 1) % 3)

        if rem:
            base = base0 + n_full * _CH
            pltpu.sync_copy(dst_hbm.at[pl.ds(base, rem)], didx_t)
            pltpu.sync_copy(e_hbm.at[pl.ds(base, rem)], rows_t)
            pltpu.sync_copy(rows_t, acc.at[didx_t], add=True)
        plsc.subcore_barrier()

        pltpu.sync_copy(acc.at[pl.ds(sid * rows_per_s, rows_per_s)],
                        out_hbm.at[cid, pl.ds(sid * rows_per_s, rows_per_s)])

    return scatter


def _row2(v):
    return v.reshape(1, -1)


def kernel(x, edge_index, edge_attr, params):
    n, node_f = x.shape
    e_cnt = edge_index.shape[1]
    latent = params['enc_node']['ln'][0].shape[0]
    n_pad = 10240
    bn = 2000
    be = 3200

    src = edge_index[0]
    dst = edge_index[1]

    (w1n, b1n), (w2n, b2n) = params['enc_node']['mlp']
    gn, ben = params['enc_node']['ln']
    xh = _mlp_ln_call(x, w1n, _row2(b1n), w2n, _row2(b2n), _row2(gn), _row2(ben), bn)

    (w1e, b1e), (w2e, b2e) = params['enc_edge']['mlp']
    ge, bee = params['enc_edge']['ln']
    eh = _mlp_ln_call(edge_attr, w1e, _row2(b1e), w2e, _row2(b2e), _row2(ge),
                      _row2(bee), be)

    sc_gather = _make_sc_gather(n, e_cnt, latent)
    sc_scatter = _make_sc_scatter(n_pad, e_cnt, latent)

    edge_grid = (e_cnt // be,)
    node_grid = (n // bn,)

    edge_call = pl.pallas_call(
        _edge_body,
        grid=edge_grid,
        in_specs=[_rows(be, (latent,)), _rows(be, (latent,)), _rows(be, (latent,)),
                  _full((latent, latent)), _full((latent, latent)),
                  _full((latent, latent)), _full((1, latent)),
                  _full((latent, latent)), _full((1, latent)),
                  _full((1, latent)), _full((1, latent))],
        out_specs=_rows(be, (latent,)),
        out_shape=jax.ShapeDtypeStruct((e_cnt, latent), F32),
    )

    node_call = pl.pallas_call(
        _node_body,
        grid=node_grid,
        in_specs=[_rows(bn, (latent,)),
                  pl.BlockSpec((2, bn, latent), lambda i: (0, i, 0)),
                  _full((latent, latent)), _full((latent, latent)),
                  _full((1, latent)), _full((latent, latent)), _full((1, latent)),
                  _full((1, latent)), _full((1, latent))],
        out_specs=_rows(bn, (latent,)),
        out_shape=jax.ShapeDtypeStruct((n, latent), F32),
    )

    for p in params['proc']:
        (ew1, eb1), (ew2, eb2) = p['edge_mlp']
        elg, elb = p['edge_ln']
        (nw1, nb1), (nw2, nb2) = p['node_mlp']
        nlg, nlb = p['node_ln']
        a_m = ew1[:latent]
        b_m = ew1[latent:2 * latent]
        c_m = ew1[2 * latent:]
        p_m = nw1[:latent]
        q_m = nw1[latent:]

        gs, gd = sc_gather(xh, src, dst)
        eh = edge_call(eh, gs, gd, a_m, b_m, c_m, _row2(eb1), ew2, _row2(eb2),
                       _row2(elg), _row2(elb))
        agg = sc_scatter(eh, dst)
        xh = node_call(xh, agg, p_m, q_m, _row2(nb1), nw2, _row2(nb2),
                       _row2(nlg), _row2(nlb))

    (dw1, db1), (dw2, db2) = params['dec']
    out_dim = dw2.shape[1]
    pad = 8 - out_dim
    dw2p = jnp.pad(dw2, ((0, 0), (0, pad)))
    db2p = jnp.pad(db2, ((0, pad),))
    dec = pl.pallas_call(
        _dec_body,
        grid=node_grid,
        in_specs=[_rows(bn, (latent,)), _full((latent, latent)), _full((1, latent)),
                  _full((latent, 8)), _full((1, 8))],
        out_specs=_rows(bn, (8,)),
        out_shape=jax.ShapeDtypeStruct((n, 8), F32),
    )(xh, dw1, _row2(db1), dw2p, _row2(db2p))
    return dec[:, :out_dim]


# speedup vs baseline: 2.7667x; 1.1072x over previous
import functools

import jax
import jax.numpy as jnp
from jax import lax
from jax.experimental import pallas as pl
from jax.experimental.pallas import tpu as pltpu
from jax.experimental.pallas import tpu_sc as plsc

F32 = jnp.float32
LN_EPS = 1e-5


def _ln(y, g, b):
    mu = jnp.mean(y, axis=-1, keepdims=True)
    var = jnp.mean((y - mu) ** 2, axis=-1, keepdims=True)
    return (y - mu) * lax.rsqrt(var + LN_EPS) * g + b


def _mlp_ln_body(x_ref, w1_ref, b1_ref, w2_ref, b2_ref, g_ref, be_ref, o_ref):
    h = jnp.maximum(
        jnp.dot(x_ref[...], w1_ref[...], preferred_element_type=F32) + b1_ref[...], 0.0)
    y = jnp.dot(h, w2_ref[...], preferred_element_type=F32) + b2_ref[...]
    o_ref[...] = _ln(y, g_ref[...], be_ref[...])


def _edge_body(e_ref, g_ref, a_ref, b1_ref, w2_ref, b2_ref, lg_ref, lb_ref,
               o_ref):
    pre = (jnp.dot(e_ref[...], a_ref[...], preferred_element_type=F32)
           + g_ref[...] + b1_ref[...])
    h = jnp.maximum(pre, 0.0)
    y = jnp.dot(h, w2_ref[...], preferred_element_type=F32) + b2_ref[...]
    o_ref[...] = e_ref[...] + _ln(y, lg_ref[...], lb_ref[...])


def _node_body(x_ref, agg_ref, p_ref, q_ref, b1_ref, w2_ref, b2_ref, lg_ref,
               lb_ref, bn_ref, cn_ref, o_ref, xb_ref, xc_ref):
    agg = agg_ref[0] + agg_ref[1]
    pre = (jnp.dot(x_ref[...], p_ref[...], preferred_element_type=F32)
           + jnp.dot(agg, q_ref[...], preferred_element_type=F32)
           + b1_ref[...])
    h = jnp.maximum(pre, 0.0)
    y = jnp.dot(h, w2_ref[...], preferred_element_type=F32) + b2_ref[...]
    xn = x_ref[...] + _ln(y, lg_ref[...], lb_ref[...])
    o_ref[...] = xn
    xb_ref[...] = jnp.dot(xn, bn_ref[...], preferred_element_type=F32)
    xc_ref[...] = jnp.dot(xn, cn_ref[...], preferred_element_type=F32)


def _node_last_body(x_ref, agg_ref, p_ref, q_ref, b1_ref, w2_ref, b2_ref,
                    lg_ref, lb_ref, o_ref):
    agg = agg_ref[0] + agg_ref[1]
    pre = (jnp.dot(x_ref[...], p_ref[...], preferred_element_type=F32)
           + jnp.dot(agg, q_ref[...], preferred_element_type=F32)
           + b1_ref[...])
    h = jnp.maximum(pre, 0.0)
    y = jnp.dot(h, w2_ref[...], preferred_element_type=F32) + b2_ref[...]
    o_ref[...] = x_ref[...] + _ln(y, lg_ref[...], lb_ref[...])


def _tables_body(x_ref, bn_ref, cn_ref, xb_ref, xc_ref):
    xn = x_ref[...]
    xb_ref[...] = jnp.dot(xn, bn_ref[...], preferred_element_type=F32)
    xc_ref[...] = jnp.dot(xn, cn_ref[...], preferred_element_type=F32)


def _dec_body(x_ref, w1_ref, b1_ref, w2_ref, b2_ref, o_ref):
    h = jnp.maximum(
        jnp.dot(x_ref[...], w1_ref[...], preferred_element_type=F32) + b1_ref[...], 0.0)
    o_ref[...] = jnp.dot(h, w2_ref[...], preferred_element_type=F32) + b2_ref[...]


def _full(shape):
    return pl.BlockSpec(shape, lambda i: (0,) * len(shape))


def _rows(bs, shape_rest):
    return pl.BlockSpec((bs,) + shape_rest, lambda i: (i,) + (0,) * len(shape_rest))


def _mlp_ln_call(x, w1, b1, w2, b2, g, be, block):
    n, fin = x.shape
    fout = w2.shape[1]
    grid = (n // block,)
    return pl.pallas_call(
        _mlp_ln_body,
        grid=grid,
        in_specs=[_rows(block, (fin,)), _full(w1.shape), _full(b1.shape),
                  _full(w2.shape), _full(b2.shape), _full(g.shape), _full(be.shape)],
        out_specs=_rows(block, (fout,)),
        out_shape=jax.ShapeDtypeStruct((n, fout), F32),
    )(x, w1, b1, w2, b2, g, be)


_CH = 128


@functools.lru_cache(maxsize=None)
def _make_sc_gather(n_nodes, n_edges, feat):
    info = plsc.get_sparse_core_info()
    nw = info.num_cores * info.num_subcores
    per_w = n_edges // nw
    assert n_edges % nw == 0
    n_full = per_w // _CH
    rem = per_w - n_full * _CH
    assert n_full >= 4 and n_full % 2 == 0
    mesh = plsc.VectorSubcoreMesh(core_axis_name="c", subcore_axis_name="s")

    scratch = [
        pltpu.VMEM((2, _CH), jnp.int32), pltpu.VMEM((2, _CH), jnp.int32),
        pltpu.VMEM((2, _CH, feat), F32), pltpu.VMEM((2, _CH, feat), F32),
    ] + [pltpu.SemaphoreType.DMA] * 12
    if rem:
        scratch += [
            pltpu.VMEM((rem,), jnp.int32), pltpu.VMEM((rem,), jnp.int32),
            pltpu.VMEM((rem, feat), F32), pltpu.VMEM((rem, feat), F32),
        ]

    @functools.partial(
        pl.kernel,
        out_type=jax.ShapeDtypeStruct((n_edges, feat), F32),
        mesh=mesh,
        scratch_types=scratch,
        compiler_params=pltpu.CompilerParams(use_tc_tiling_on_sc=False),
    )
    def gather(xb_hbm, xc_hbm, src_hbm, dst_hbm, g_hbm, *refs):
        sidx, didx, ra, rb = refs[0:4]
        semis = refs[4:6]
        semid = refs[6:8]
        semga = refs[8:10]
        semgb = refs[10:12]
        semwa = refs[12:14]
        if rem:
            sidx_t, didx_t, rows_at, rows_bt = refs[16:20]
        wid = lax.axis_index("s") * info.num_cores + lax.axis_index("c")
        base0 = wid * per_w

        def i_issue(c, s):
            pltpu.async_copy(src_hbm.at[pl.ds(base0 + c * _CH, _CH)],
                             sidx.at[s], semis[s])
            pltpu.async_copy(dst_hbm.at[pl.ds(base0 + c * _CH, _CH)],
                             didx.at[s], semid[s])

        def i_wait(s):
            pltpu.make_async_copy(src_hbm.at[pl.ds(0, _CH)], sidx.at[s],
                                  semis[s]).wait()
            pltpu.make_async_copy(dst_hbm.at[pl.ds(0, _CH)], didx.at[s],
                                  semid[s]).wait()

        def g_issue(s):
            pltpu.async_copy(xb_hbm.at[sidx.at[s]], ra.at[s], semga[s])
            pltpu.async_copy(xc_hbm.at[didx.at[s]], rb.at[s], semgb[s])

        def g_wait(s):
            pltpu.make_async_copy(g_hbm.at[pl.ds(0, _CH)], ra.at[s],
                                  semga[s]).wait()
            pltpu.make_async_copy(g_hbm.at[pl.ds(0, _CH)], rb.at[s],
                                  semgb[s]).wait()

        def add_rows(s):
            def body(r, carry):
                for c in range(feat // 16):
                    ra[s, r, pl.ds(c * 16, 16)] = (
                        ra[s, r, pl.ds(c * 16, 16)]
                        + rb[s, r, pl.ds(c * 16, 16)])
                return carry

            lax.fori_loop(0, _CH, body, 0, unroll=4)

        def w_issue(c, s):
            pltpu.async_copy(ra.at[s], g_hbm.at[pl.ds(base0 + c * _CH, _CH)],
                             semwa[s])

        def w_wait(s):
            pltpu.make_async_copy(ra.at[s], g_hbm.at[pl.ds(0, _CH)],
                                  semwa[s]).wait()

        def iteration(j, s, do_next_gather, do_write_wait, do_idx_prefetch):
            s1 = 1 - s
            if do_next_gather:
                i_wait(s1)
                if do_write_wait:
                    w_wait(s1)
                g_issue(s1)
            g_wait(s)
            add_rows(s)
            w_issue(j, s)
            if do_idx_prefetch:
                i_issue(j + 2, s)

        i_issue(0, 0)
        i_issue(1, 1)
        i_wait(0)
        g_issue(0)
        iteration(0, 0, True, False, True)
        def body(g, carry):
            j = 2 * g + 1
            iteration(j, 1, True, True, True)
            iteration(j + 1, 0, True, True, True)
            return carry

        lax.fori_loop(0, n_full // 2 - 2, body, 0)
        iteration(n_full - 3, 1, True, True, True)
        iteration(n_full - 2, 0, True, True, False)
        iteration(n_full - 1, 1, False, False, False)
        w_wait(0)
        w_wait(1)

        if rem:
            base = base0 + n_full * _CH
            pltpu.sync_copy(src_hbm.at[pl.ds(base, rem)], sidx_t)
            pltpu.sync_copy(dst_hbm.at[pl.ds(base, rem)], didx_t)
            ca = pltpu.async_copy(xb_hbm.at[sidx_t], rows_at, semga[0])
            cb = pltpu.async_copy(xc_hbm.at[didx_t], rows_bt, semgb[0])
            ca.wait()
            cb.wait()

            def body_t(r, carry):
                for c in range(feat // 16):
                    rows_at[r, pl.ds(c * 16, 16)] = (
                        rows_at[r, pl.ds(c * 16, 16)]
                        + rows_bt[r, pl.ds(c * 16, 16)])
                return carry

            lax.fori_loop(0, rem, body_t, 0, unroll=4)
            pltpu.sync_copy(rows_at, g_hbm.at[pl.ds(base, rem)])

    return gather


@functools.lru_cache(maxsize=None)
def _make_sc_scatter(n_nodes_pad, n_edges, feat):
    info = plsc.get_sparse_core_info()
    nc, ns = info.num_cores, info.num_subcores
    nw = nc * ns
    per_w = n_edges // nw
    assert n_edges % nw == 0
    n_full = per_w // _CH
    rem = per_w - n_full * _CH
    rows_per_s = n_nodes_pad // ns
    assert n_nodes_pad % (ns * _CH) == 0
    zgroups = rows_per_s // _CH
    mesh = plsc.VectorSubcoreMesh(core_axis_name="c", subcore_axis_name="s")

    assert n_full >= 6 and (n_full - 3) % 3 == 0
    scratch = [
        pltpu.VMEM((3, _CH), jnp.int32),
        pltpu.VMEM((3, _CH, feat), F32),
        pltpu.VMEM_SHARED((n_nodes_pad, feat), F32),
    ] + [pltpu.SemaphoreType.DMA] * 9
    if rem:
        scratch += [pltpu.VMEM((rem,), jnp.int32), pltpu.VMEM((rem, feat), F32)]

    @functools.partial(
        pl.kernel,
        out_type=jax.ShapeDtypeStruct((nc, n_nodes_pad, feat), F32),
        mesh=mesh,
        scratch_types=scratch,
        compiler_params=pltpu.CompilerParams(use_tc_tiling_on_sc=False),
    )
    def scatter(e_hbm, dst_hbm, out_hbm, *refs):
        didx, rows, acc = refs[0:3]
        semli = refs[3:6]
        semlr = refs[6:9]
        semsc = refs[9:12]
        if rem:
            didx_t, rows_t = refs[12:14]
        cid = lax.axis_index("c")
        sid = lax.axis_index("s")
        wid = sid * nc + cid
        base0 = wid * per_w

        def zrow(i, carry):
            for c in range(feat // 16):
                rows[0, i, pl.ds(c * 16, 16)] = jnp.zeros((16,), F32)
            return carry

        lax.fori_loop(0, _CH, zrow, 0)
        for t in range(zgroups):
            pltpu.sync_copy(rows.at[0],
                            acc.at[pl.ds(sid * rows_per_s + t * _CH, _CH)])
        plsc.subcore_barrier()

        def l_issue(c, s):
            pltpu.async_copy(dst_hbm.at[pl.ds(base0 + c * _CH, _CH)],
                             didx.at[s], semli[s])
            pltpu.async_copy(e_hbm.at[pl.ds(base0 + c * _CH, _CH)],
                             rows.at[s], semlr[s])

        def l_wait(s):
            pltpu.make_async_copy(dst_hbm.at[pl.ds(0, _CH)], didx.at[s],
                                  semli[s]).wait()
            pltpu.make_async_copy(e_hbm.at[pl.ds(0, _CH)], rows.at[s],
                                  semlr[s]).wait()

        def a_issue(s):
            pltpu.async_copy(rows.at[s], acc.at[didx.at[s]], semsc[s],
                             add=True)

        def a_wait(s):
            pltpu.make_async_copy(rows.at[s], acc.at[didx.at[s]],
                                  semsc[s]).wait()

        def iteration(j, s, do_scatter_wait, do_prefetch):
            l_wait(s)
            a_issue(s)
            if do_scatter_wait:
                a_wait((s + 2) % 3)
            if do_prefetch:
                l_issue(j + 2, (s + 2) % 3)

        l_issue(0, 0)
        l_issue(1, 1)
        l_issue(2, 2)
        iteration(0, 0, False, False)
        def body(g, carry):
            j = 3 * g + 1
            iteration(j, 1, True, True)
            iteration(j + 1, 2, True, True)
            iteration(j + 2, 0, True, True)
            return carry

        lax.fori_loop(0, (n_full - 3) // 3, body, 0)
        j0 = n_full - 2
        iteration(j0, j0 % 3, True, False)
        iteration(j0 + 1, (j0 + 1) % 3, True, False)
        a_wait((j0 + 1) % 3)

        if rem:
            base = base0 + n_full * _CH
            pltpu.sync_copy(dst_hbm.at[pl.ds(base, rem)], didx_t)
            pltpu.sync_copy(e_hbm.at[pl.ds(base, rem)], rows_t)
            pltpu.sync_copy(rows_t, acc.at[didx_t], add=True)
        plsc.subcore_barrier()

        pltpu.sync_copy(acc.at[pl.ds(sid * rows_per_s, rows_per_s)],
                        out_hbm.at[cid, pl.ds(sid * rows_per_s, rows_per_s)])

    return scatter


def _row2(v):
    return v.reshape(1, -1)


def kernel(x, edge_index, edge_attr, params):
    n, node_f = x.shape
    e_cnt = edge_index.shape[1]
    latent = params['enc_node']['ln'][0].shape[0]
    n_pad = 10240
    bn = 2000
    be = 3200

    src = edge_index[0]
    dst = edge_index[1]

    (w1n, b1n), (w2n, b2n) = params['enc_node']['mlp']
    gn, ben = params['enc_node']['ln']
    xh = _mlp_ln_call(x, w1n, _row2(b1n), w2n, _row2(b2n), _row2(gn), _row2(ben), bn)

    (w1e, b1e), (w2e, b2e) = params['enc_edge']['mlp']
    ge, bee = params['enc_edge']['ln']
    eh = _mlp_ln_call(edge_attr, w1e, _row2(b1e), w2e, _row2(b2e), _row2(ge),
                      _row2(bee), be)

    sc_gather = _make_sc_gather(n, e_cnt, latent)
    sc_scatter = _make_sc_scatter(n_pad, e_cnt, latent)

    edge_grid = (e_cnt // be,)
    node_grid = (n // bn,)

    edge_call = pl.pallas_call(
        _edge_body,
        grid=edge_grid,
        in_specs=[_rows(be, (latent,)), _rows(be, (latent,)),
                  _full((latent, latent)), _full((1, latent)),
                  _full((latent, latent)), _full((1, latent)),
                  _full((1, latent)), _full((1, latent))],
        out_specs=_rows(be, (latent,)),
        out_shape=jax.ShapeDtypeStruct((e_cnt, latent), F32),
    )

    node_call = pl.pallas_call(
        _node_body,
        grid=node_grid,
        in_specs=[_rows(bn, (latent,)),
                  pl.BlockSpec((2, bn, latent), lambda i: (0, i, 0)),
                  _full((latent, latent)), _full((latent, latent)),
                  _full((1, latent)), _full((latent, latent)), _full((1, latent)),
                  _full((1, latent)), _full((1, latent)),
                  _full((latent, latent)), _full((latent, latent))],
        out_specs=[_rows(bn, (latent,))] * 3,
        out_shape=[jax.ShapeDtypeStruct((n, latent), F32)] * 3,
    )

    node_last_call = pl.pallas_call(
        _node_last_body,
        grid=node_grid,
        in_specs=[_rows(bn, (latent,)),
                  pl.BlockSpec((2, bn, latent), lambda i: (0, i, 0)),
                  _full((latent, latent)), _full((latent, latent)),
                  _full((1, latent)), _full((latent, latent)), _full((1, latent)),
                  _full((1, latent)), _full((1, latent))],
        out_specs=_rows(bn, (latent,)),
        out_shape=jax.ShapeDtypeStruct((n, latent), F32),
    )

    tables_call = pl.pallas_call(
        _tables_body,
        grid=node_grid,
        in_specs=[_rows(bn, (latent,)), _full((latent, latent)),
                  _full((latent, latent))],
        out_specs=[_rows(bn, (latent,))] * 2,
        out_shape=[jax.ShapeDtypeStruct((n, latent), F32)] * 2,
    )

    def edge_w(p):
        (ew1, _), _ = p['edge_mlp']
        return ew1[latent:2 * latent], ew1[2 * latent:]

    b0, c0 = edge_w(params['proc'][0])
    xb, xc = tables_call(xh, b0, c0)

    n_steps = len(params['proc'])
    for s, p in enumerate(params['proc']):
        (ew1, eb1), (ew2, eb2) = p['edge_mlp']
        elg, elb = p['edge_ln']
        (nw1, nb1), (nw2, nb2) = p['node_mlp']
        nlg, nlb = p['node_ln']
        a_m = ew1[:latent]
        p_m = nw1[:latent]
        q_m = nw1[latent:]

        g = sc_gather(xb, xc, src, dst)
        eh = edge_call(eh, g, a_m, _row2(eb1), ew2, _row2(eb2),
                       _row2(elg), _row2(elb))
        agg = sc_scatter(eh, dst)
        if s + 1 < n_steps:
            bn_m, cn_m = edge_w(params['proc'][s + 1])
            xh, xb, xc = node_call(xh, agg, p_m, q_m, _row2(nb1), nw2,
                                   _row2(nb2), _row2(nlg), _row2(nlb),
                                   bn_m, cn_m)
        else:
            xh = node_last_call(xh, agg, p_m, q_m, _row2(nb1), nw2,
                                _row2(nb2), _row2(nlg), _row2(nlb))

    (dw1, db1), (dw2, db2) = params['dec']
    out_dim = dw2.shape[1]
    pad = 8 - out_dim
    dw2p = jnp.pad(dw2, ((0, 0), (0, pad)))
    db2p = jnp.pad(db2, ((0, pad),))
    dec = pl.pallas_call(
        _dec_body,
        grid=node_grid,
        in_specs=[_rows(bn, (latent,)), _full((latent, latent)), _full((1, latent)),
                  _full((latent, 8)), _full((1, 8))],
        out_specs=_rows(bn, (8,)),
        out_shape=jax.ShapeDtypeStruct((n, 8), F32),
    )(xh, dw1, _row2(db1), dw2p, _row2(db2p))
    return dec[:, :out_dim]


# speedup vs baseline: 3.1889x; 1.1526x over previous
import functools

import jax
import jax.numpy as jnp
from jax import lax
from jax.experimental import pallas as pl
from jax.experimental.pallas import tpu as pltpu
from jax.experimental.pallas import tpu_sc as plsc

F32 = jnp.float32
LN_EPS = 1e-5


def _ln(y, g, b):
    mu = jnp.mean(y, axis=-1, keepdims=True)
    var = jnp.mean((y - mu) ** 2, axis=-1, keepdims=True)
    return (y - mu) * lax.rsqrt(var + LN_EPS) * g + b


def _mlp_ln_body(x_ref, w1_ref, b1_ref, w2_ref, b2_ref, g_ref, be_ref, o_ref):
    h = jnp.maximum(
        jnp.dot(x_ref[...], w1_ref[...], preferred_element_type=F32) + b1_ref[...], 0.0)
    y = jnp.dot(h, w2_ref[...], preferred_element_type=F32) + b2_ref[...]
    o_ref[...] = _ln(y, g_ref[...], be_ref[...])


def _edge_body(e_ref, g_ref, a_ref, b1_ref, w2_ref, b2_ref, lg_ref, lb_ref,
               o_ref):
    pre = (jnp.dot(e_ref[...], a_ref[...], preferred_element_type=F32)
           + g_ref[...] + b1_ref[...])
    h = jnp.maximum(pre, 0.0)
    y = jnp.dot(h, w2_ref[...], preferred_element_type=F32) + b2_ref[...]
    o_ref[...] = e_ref[...] + _ln(y, lg_ref[...], lb_ref[...])


def _node_body(x_ref, agg_ref, p_ref, q_ref, b1_ref, w2_ref, b2_ref, lg_ref,
               lb_ref, bn_ref, cn_ref, o_ref, xb_ref, xc_ref):
    agg = agg_ref[0] + agg_ref[1]
    pre = (jnp.dot(x_ref[...], p_ref[...], preferred_element_type=F32)
           + jnp.dot(agg, q_ref[...], preferred_element_type=F32)
           + b1_ref[...])
    h = jnp.maximum(pre, 0.0)
    y = jnp.dot(h, w2_ref[...], preferred_element_type=F32) + b2_ref[...]
    xn = x_ref[...] + _ln(y, lg_ref[...], lb_ref[...])
    o_ref[...] = xn
    xb_ref[...] = jnp.dot(xn, bn_ref[...], preferred_element_type=F32)
    xc_ref[...] = jnp.dot(xn, cn_ref[...], preferred_element_type=F32)


def _node_last_body(x_ref, agg_ref, p_ref, q_ref, b1_ref, w2_ref, b2_ref,
                    lg_ref, lb_ref, o_ref):
    agg = agg_ref[0] + agg_ref[1]
    pre = (jnp.dot(x_ref[...], p_ref[...], preferred_element_type=F32)
           + jnp.dot(agg, q_ref[...], preferred_element_type=F32)
           + b1_ref[...])
    h = jnp.maximum(pre, 0.0)
    y = jnp.dot(h, w2_ref[...], preferred_element_type=F32) + b2_ref[...]
    o_ref[...] = x_ref[...] + _ln(y, lg_ref[...], lb_ref[...])


def _tables_body(x_ref, bn_ref, cn_ref, xb_ref, xc_ref):
    xn = x_ref[...]
    xb_ref[...] = jnp.dot(xn, bn_ref[...], preferred_element_type=F32)
    xc_ref[...] = jnp.dot(xn, cn_ref[...], preferred_element_type=F32)


def _dec_body(x_ref, w1_ref, b1_ref, w2_ref, b2_ref, o_ref):
    h = jnp.maximum(
        jnp.dot(x_ref[...], w1_ref[...], preferred_element_type=F32) + b1_ref[...], 0.0)
    o_ref[...] = jnp.dot(h, w2_ref[...], preferred_element_type=F32) + b2_ref[...]


def _full(shape):
    return pl.BlockSpec(shape, lambda i: (0,) * len(shape))


def _rows(bs, shape_rest):
    return pl.BlockSpec((bs,) + shape_rest, lambda i: (i,) + (0,) * len(shape_rest))


def _mlp_ln_call(x, w1, b1, w2, b2, g, be, block):
    n, fin = x.shape
    fout = w2.shape[1]
    grid = (n // block,)
    return pl.pallas_call(
        _mlp_ln_body,
        grid=grid,
        in_specs=[_rows(block, (fin,)), _full(w1.shape), _full(b1.shape),
                  _full(w2.shape), _full(b2.shape), _full(g.shape), _full(be.shape)],
        out_specs=_rows(block, (fout,)),
        out_shape=jax.ShapeDtypeStruct((n, fout), F32),
    )(x, w1, b1, w2, b2, g, be)


_CH = 128


@functools.lru_cache(maxsize=None)
def _make_sc_gather(n_nodes, n_edges, feat):
    info = plsc.get_sparse_core_info()
    nw = info.num_cores * info.num_subcores
    per_w = n_edges // nw
    assert n_edges % nw == 0
    n_full = per_w // _CH
    rem = per_w - n_full * _CH
    assert n_full >= 4 and n_full % 2 == 0
    mesh = plsc.VectorSubcoreMesh(core_axis_name="c", subcore_axis_name="s")

    scratch = [
        pltpu.VMEM((2, _CH), jnp.int32), pltpu.VMEM((2, _CH), jnp.int32),
        pltpu.VMEM((2, _CH, feat), F32), pltpu.VMEM((2, _CH, feat), F32),
    ] + [pltpu.SemaphoreType.DMA] * 12
    if rem:
        scratch += [
            pltpu.VMEM((rem,), jnp.int32), pltpu.VMEM((rem,), jnp.int32),
            pltpu.VMEM((rem, feat), F32), pltpu.VMEM((rem, feat), F32),
        ]

    @functools.partial(
        pl.kernel,
        out_type=jax.ShapeDtypeStruct((n_edges, feat), F32),
        mesh=mesh,
        scratch_types=scratch,
        compiler_params=pltpu.CompilerParams(use_tc_tiling_on_sc=False),
    )
    def gather(xb_hbm, xc_hbm, src_hbm, dst_hbm, g_hbm, *refs):
        sidx, didx, ra, rb = refs[0:4]
        semis = refs[4:6]
        semid = refs[6:8]
        semga = refs[8:10]
        semgb = refs[10:12]
        semwa = refs[12:14]
        if rem:
            sidx_t, didx_t, rows_at, rows_bt = refs[16:20]
        wid = lax.axis_index("s") * info.num_cores + lax.axis_index("c")
        base0 = wid * per_w

        def i_issue(c, s):
            pltpu.async_copy(src_hbm.at[pl.ds(base0 + c * _CH, _CH)],
                             sidx.at[s], semis[s])
            pltpu.async_copy(dst_hbm.at[pl.ds(base0 + c * _CH, _CH)],
                             didx.at[s], semid[s])

        def i_wait(s):
            pltpu.make_async_copy(src_hbm.at[pl.ds(0, _CH)], sidx.at[s],
                                  semis[s]).wait()
            pltpu.make_async_copy(dst_hbm.at[pl.ds(0, _CH)], didx.at[s],
                                  semid[s]).wait()

        def g_issue(s):
            pltpu.async_copy(xb_hbm.at[sidx.at[s]], ra.at[s], semga[s])
            pltpu.async_copy(xc_hbm.at[didx.at[s]], rb.at[s], semgb[s])

        def g_wait(s):
            pltpu.make_async_copy(g_hbm.at[pl.ds(0, _CH)], ra.at[s],
                                  semga[s]).wait()
            pltpu.make_async_copy(g_hbm.at[pl.ds(0, _CH)], rb.at[s],
                                  semgb[s]).wait()

        def add_rows(s):
            def body(r, carry):
                for c in range(feat // 16):
                    plsc.addupdate(ra.at[s, r, pl.ds(c * 16, 16)],
                                   rb[s, r, pl.ds(c * 16, 16)])
                return carry

            lax.fori_loop(0, _CH, body, 0, unroll=4)

        def w_issue(c, s):
            pltpu.async_copy(ra.at[s], g_hbm.at[pl.ds(base0 + c * _CH, _CH)],
                             semwa[s])

        def w_wait(s):
            pltpu.make_async_copy(ra.at[s], g_hbm.at[pl.ds(0, _CH)],
                                  semwa[s]).wait()

        def iteration(j, s, do_next_gather, do_write_wait, do_idx_prefetch):
            s1 = 1 - s
            if do_next_gather:
                i_wait(s1)
                if do_write_wait:
                    w_wait(s1)
                g_issue(s1)
            g_wait(s)
            add_rows(s)
            w_issue(j, s)
            if do_idx_prefetch:
                i_issue(j + 2, s)

        i_issue(0, 0)
        i_issue(1, 1)
        i_wait(0)
        g_issue(0)
        iteration(0, 0, True, False, True)
        def body(g, carry):
            j = 2 * g + 1
            iteration(j, 1, True, True, True)
            iteration(j + 1, 0, True, True, True)
            return carry

        lax.fori_loop(0, n_full // 2 - 2, body, 0)
        iteration(n_full - 3, 1, True, True, True)
        iteration(n_full - 2, 0, True, True, False)
        iteration(n_full - 1, 1, False, False, False)
        w_wait(0)
        w_wait(1)

        if rem:
            base = base0 + n_full * _CH
            pltpu.sync_copy(src_hbm.at[pl.ds(base, rem)], sidx_t)
            pltpu.sync_copy(dst_hbm.at[pl.ds(base, rem)], didx_t)
            ca = pltpu.async_copy(xb_hbm.at[sidx_t], rows_at, semga[0])
            cb = pltpu.async_copy(xc_hbm.at[didx_t], rows_bt, semgb[0])
            ca.wait()
            cb.wait()

            def body_t(r, carry):
                for c in range(feat // 16):
                    plsc.addupdate(rows_at.at[r, pl.ds(c * 16, 16)],
                                   rows_bt[r, pl.ds(c * 16, 16)])
                return carry

            lax.fori_loop(0, rem, body_t, 0, unroll=4)
            pltpu.sync_copy(rows_at, g_hbm.at[pl.ds(base, rem)])

    return gather


@functools.lru_cache(maxsize=None)
def _make_sc_scatter(n_nodes_pad, n_edges, feat):
    info = plsc.get_sparse_core_info()
    nc, ns = info.num_cores, info.num_subcores
    nw = nc * ns
    per_w = n_edges // nw
    assert n_edges % nw == 0
    n_full = per_w // _CH
    rem = per_w - n_full * _CH
    rows_per_s = n_nodes_pad // ns
    assert n_nodes_pad % (ns * _CH) == 0
    zgroups = rows_per_s // _CH
    mesh = plsc.VectorSubcoreMesh(core_axis_name="c", subcore_axis_name="s")

    assert n_full >= 6 and (n_full - 3) % 3 == 0
    scratch = [
        pltpu.VMEM((3, _CH), jnp.int32),
        pltpu.VMEM((3, _CH, feat), F32),
        pltpu.VMEM_SHARED((n_nodes_pad, feat), F32),
    ] + [pltpu.SemaphoreType.DMA] * 9
    if rem:
        scratch += [pltpu.VMEM((rem,), jnp.int32), pltpu.VMEM((rem, feat), F32)]

    @functools.partial(
        pl.kernel,
        out_type=jax.ShapeDtypeStruct((nc, n_nodes_pad, feat), F32),
        mesh=mesh,
        scratch_types=scratch,
        compiler_params=pltpu.CompilerParams(use_tc_tiling_on_sc=False),
    )
    def scatter(e_hbm, dst_hbm, out_hbm, *refs):
        didx, rows, acc = refs[0:3]
        semli = refs[3:6]
        semlr = refs[6:9]
        semsc = refs[9:12]
        if rem:
            didx_t, rows_t = refs[12:14]
        cid = lax.axis_index("c")
        sid = lax.axis_index("s")
        wid = sid * nc + cid
        base0 = wid * per_w

        def zrow(i, carry):
            for c in range(feat // 16):
                rows[0, i, pl.ds(c * 16, 16)] = jnp.zeros((16,), F32)
            return carry

        lax.fori_loop(0, _CH, zrow, 0)
        for t in range(zgroups):
            pltpu.sync_copy(rows.at[0],
                            acc.at[pl.ds(sid * rows_per_s + t * _CH, _CH)])
        plsc.subcore_barrier()

        def l_issue(c, s):
            pltpu.async_copy(dst_hbm.at[pl.ds(base0 + c * _CH, _CH)],
                             didx.at[s], semli[s])
            pltpu.async_copy(e_hbm.at[pl.ds(base0 + c * _CH, _CH)],
                             rows.at[s], semlr[s])

        def l_wait(s):
            pltpu.make_async_copy(dst_hbm.at[pl.ds(0, _CH)], didx.at[s],
                                  semli[s]).wait()
            pltpu.make_async_copy(e_hbm.at[pl.ds(0, _CH)], rows.at[s],
                                  semlr[s]).wait()

        def a_issue(s):
            pltpu.async_copy(rows.at[s], acc.at[didx.at[s]], semsc[s],
                             add=True)

        def a_wait(s):
            pltpu.make_async_copy(rows.at[s], acc.at[didx.at[s]],
                                  semsc[s]).wait()

        def iteration(j, s, do_scatter_wait, do_prefetch):
            l_wait(s)
            a_issue(s)
            if do_scatter_wait:
                a_wait((s + 2) % 3)
            if do_prefetch:
                l_issue(j + 2, (s + 2) % 3)

        l_issue(0, 0)
        l_issue(1, 1)
        l_issue(2, 2)
        iteration(0, 0, False, False)
        def body(g, carry):
            j = 3 * g + 1
            iteration(j, 1, True, True)
            iteration(j + 1, 2, True, True)
            iteration(j + 2, 0, True, True)
            return carry

        lax.fori_loop(0, (n_full - 3) // 3, body, 0)
        j0 = n_full - 2
        iteration(j0, j0 % 3, True, False)
        iteration(j0 + 1, (j0 + 1) % 3, True, False)
        a_wait((j0 + 1) % 3)

        if rem:
            base = base0 + n_full * _CH
            pltpu.sync_copy(dst_hbm.at[pl.ds(base, rem)], didx_t)
            pltpu.sync_copy(e_hbm.at[pl.ds(base, rem)], rows_t)
            pltpu.sync_copy(rows_t, acc.at[didx_t], add=True)
        plsc.subcore_barrier()

        pltpu.sync_copy(acc.at[pl.ds(sid * rows_per_s, rows_per_s)],
                        out_hbm.at[cid, pl.ds(sid * rows_per_s, rows_per_s)])

    return scatter


def _row2(v):
    return v.reshape(1, -1)


def kernel(x, edge_index, edge_attr, params):
    n, node_f = x.shape
    e_cnt = edge_index.shape[1]
    latent = params['enc_node']['ln'][0].shape[0]
    n_pad = 10240
    bn = 2000
    be = 3200

    src = edge_index[0]
    dst = edge_index[1]

    (w1n, b1n), (w2n, b2n) = params['enc_node']['mlp']
    gn, ben = params['enc_node']['ln']
    xh = _mlp_ln_call(x, w1n, _row2(b1n), w2n, _row2(b2n), _row2(gn), _row2(ben), bn)

    (w1e, b1e), (w2e, b2e) = params['enc_edge']['mlp']
    ge, bee = params['enc_edge']['ln']
    eh = _mlp_ln_call(edge_attr, w1e, _row2(b1e), w2e, _row2(b2e), _row2(ge),
                      _row2(bee), be)

    sc_gather = _make_sc_gather(n, e_cnt, latent)
    sc_scatter = _make_sc_scatter(n_pad, e_cnt, latent)

    edge_grid = (e_cnt // be,)
    node_grid = (n // bn,)

    edge_call = pl.pallas_call(
        _edge_body,
        grid=edge_grid,
        in_specs=[_rows(be, (latent,)), _rows(be, (latent,)),
                  _full((latent, latent)), _full((1, latent)),
                  _full((latent, latent)), _full((1, latent)),
                  _full((1, latent)), _full((1, latent))],
        out_specs=_rows(be, (latent,)),
        out_shape=jax.ShapeDtypeStruct((e_cnt, latent), F32),
    )

    node_call = pl.pallas_call(
        _node_body,
        grid=node_grid,
        in_specs=[_rows(bn, (latent,)),
                  pl.BlockSpec((2, bn, latent), lambda i: (0, i, 0)),
                  _full((latent, latent)), _full((latent, latent)),
                  _full((1, latent)), _full((latent, latent)), _full((1, latent)),
                  _full((1, latent)), _full((1, latent)),
                  _full((latent, latent)), _full((latent, latent))],
        out_specs=[_rows(bn, (latent,))] * 3,
        out_shape=[jax.ShapeDtypeStruct((n, latent), F32)] * 3,
    )

    node_last_call = pl.pallas_call(
        _node_last_body,
        grid=node_grid,
        in_specs=[_rows(bn, (latent,)),
                  pl.BlockSpec((2, bn, latent), lambda i: (0, i, 0)),
                  _full((latent, latent)), _full((latent, latent)),
                  _full((1, latent)), _full((latent, latent)), _full((1, latent)),
                  _full((1, latent)), _full((1, latent))],
        out_specs=_rows(bn, (latent,)),
        out_shape=jax.ShapeDtypeStruct((n, latent), F32),
    )

    tables_call = pl.pallas_call(
        _tables_body,
        grid=node_grid,
        in_specs=[_rows(bn, (latent,)), _full((latent, latent)),
                  _full((latent, latent))],
        out_specs=[_rows(bn, (latent,))] * 2,
        out_shape=[jax.ShapeDtypeStruct((n, latent), F32)] * 2,
    )

    def edge_w(p):
        (ew1, _), _ = p['edge_mlp']
        return ew1[latent:2 * latent], ew1[2 * latent:]

    b0, c0 = edge_w(params['proc'][0])
    xb, xc = tables_call(xh, b0, c0)

    n_steps = len(params['proc'])
    for s, p in enumerate(params['proc']):
        (ew1, eb1), (ew2, eb2) = p['edge_mlp']
        elg, elb = p['edge_ln']
        (nw1, nb1), (nw2, nb2) = p['node_mlp']
        nlg, nlb = p['node_ln']
        a_m = ew1[:latent]
        p_m = nw1[:latent]
        q_m = nw1[latent:]

        g = sc_gather(xb, xc, src, dst)
        eh = edge_call(eh, g, a_m, _row2(eb1), ew2, _row2(eb2),
                       _row2(elg), _row2(elb))
        agg = sc_scatter(eh, dst)
        if s + 1 < n_steps:
            bn_m, cn_m = edge_w(params['proc'][s + 1])
            xh, xb, xc = node_call(xh, agg, p_m, q_m, _row2(nb1), nw2,
                                   _row2(nb2), _row2(nlg), _row2(nlb),
                                   bn_m, cn_m)
        else:
            xh = node_last_call(xh, agg, p_m, q_m, _row2(nb1), nw2,
                                _row2(nb2), _row2(nlg), _row2(nlb))

    (dw1, db1), (dw2, db2) = params['dec']
    out_dim = dw2.shape[1]
    pad = 8 - out_dim
    dw2p = jnp.pad(dw2, ((0, 0), (0, pad)))
    db2p = jnp.pad(db2, ((0, pad),))
    dec = pl.pallas_call(
        _dec_body,
        grid=node_grid,
        in_specs=[_rows(bn, (latent,)), _full((latent, latent)), _full((1, latent)),
                  _full((latent, 8)), _full((1, 8))],
        out_specs=_rows(bn, (8,)),
        out_shape=jax.ShapeDtypeStruct((n, 8), F32),
    )(xh, dw1, _row2(db1), dw2p, _row2(db2p))
    return dec[:, :out_dim]


# speedup vs baseline: 3.3368x; 1.0464x over previous
import functools

import jax
import jax.numpy as jnp
from jax import lax
from jax.experimental import pallas as pl
from jax.experimental.pallas import tpu as pltpu
from jax.experimental.pallas import tpu_sc as plsc

F32 = jnp.float32
LN_EPS = 1e-5


def _ln(y, g, b):
    mu = jnp.mean(y, axis=-1, keepdims=True)
    var = jnp.mean((y - mu) ** 2, axis=-1, keepdims=True)
    return (y - mu) * lax.rsqrt(var + LN_EPS) * g + b


def _mlp_ln_body(x_ref, w1_ref, b1_ref, w2_ref, b2_ref, g_ref, be_ref, o_ref):
    h = jnp.maximum(
        jnp.dot(x_ref[...], w1_ref[...], preferred_element_type=F32) + b1_ref[...], 0.0)
    y = jnp.dot(h, w2_ref[...], preferred_element_type=F32) + b2_ref[...]
    o_ref[...] = _ln(y, g_ref[...], be_ref[...])


def _edge_body(e_ref, g_ref, a_ref, b1_ref, w2_ref, b2_ref, lg_ref, lb_ref,
               o_ref):
    pre = (jnp.dot(e_ref[...], a_ref[...], preferred_element_type=F32)
           + g_ref[...] + b1_ref[...])
    h = jnp.maximum(pre, 0.0)
    y = jnp.dot(h, w2_ref[...], preferred_element_type=F32) + b2_ref[...]
    o_ref[...] = e_ref[...] + _ln(y, lg_ref[...], lb_ref[...])


def _node_body(x_ref, agg_ref, agg2_ref, p_ref, q_ref, b1_ref, w2_ref, b2_ref,
               lg_ref, lb_ref, bn_ref, cn_ref, o_ref, xb_ref, xc_ref):
    agg = (agg_ref[0] + agg_ref[1]) + (agg2_ref[0] + agg2_ref[1])
    pre = (jnp.dot(x_ref[...], p_ref[...], preferred_element_type=F32)
           + jnp.dot(agg, q_ref[...], preferred_element_type=F32)
           + b1_ref[...])
    h = jnp.maximum(pre, 0.0)
    y = jnp.dot(h, w2_ref[...], preferred_element_type=F32) + b2_ref[...]
    xn = x_ref[...] + _ln(y, lg_ref[...], lb_ref[...])
    o_ref[...] = xn
    xb_ref[...] = jnp.dot(xn, bn_ref[...], preferred_element_type=F32)
    xc_ref[...] = jnp.dot(xn, cn_ref[...], preferred_element_type=F32)


def _node_last_body(x_ref, agg_ref, agg2_ref, p_ref, q_ref, b1_ref, w2_ref,
                    b2_ref, lg_ref, lb_ref, o_ref):
    agg = (agg_ref[0] + agg_ref[1]) + (agg2_ref[0] + agg2_ref[1])
    pre = (jnp.dot(x_ref[...], p_ref[...], preferred_element_type=F32)
           + jnp.dot(agg, q_ref[...], preferred_element_type=F32)
           + b1_ref[...])
    h = jnp.maximum(pre, 0.0)
    y = jnp.dot(h, w2_ref[...], preferred_element_type=F32) + b2_ref[...]
    o_ref[...] = x_ref[...] + _ln(y, lg_ref[...], lb_ref[...])


def _tables_body(x_ref, bn_ref, cn_ref, xb_ref, xc_ref):
    xn = x_ref[...]
    xb_ref[...] = jnp.dot(xn, bn_ref[...], preferred_element_type=F32)
    xc_ref[...] = jnp.dot(xn, cn_ref[...], preferred_element_type=F32)


def _dec_body(x_ref, w1_ref, b1_ref, w2_ref, b2_ref, o_ref):
    h = jnp.maximum(
        jnp.dot(x_ref[...], w1_ref[...], preferred_element_type=F32) + b1_ref[...], 0.0)
    o_ref[...] = jnp.dot(h, w2_ref[...], preferred_element_type=F32) + b2_ref[...]


def _full(shape):
    return pl.BlockSpec(shape, lambda i: (0,) * len(shape))


def _rows(bs, shape_rest):
    return pl.BlockSpec((bs,) + shape_rest, lambda i: (i,) + (0,) * len(shape_rest))


def _mlp_ln_call(x, w1, b1, w2, b2, g, be, block):
    n, fin = x.shape
    fout = w2.shape[1]
    grid = (n // block,)
    return pl.pallas_call(
        _mlp_ln_body,
        grid=grid,
        in_specs=[_rows(block, (fin,)), _full(w1.shape), _full(b1.shape),
                  _full(w2.shape), _full(b2.shape), _full(g.shape), _full(be.shape)],
        out_specs=_rows(block, (fout,)),
        out_shape=jax.ShapeDtypeStruct((n, fout), F32),
    )(x, w1, b1, w2, b2, g, be)


_CH = 128


@functools.lru_cache(maxsize=None)
def _make_sc_gather(n_nodes, n_edges, feat):
    info = plsc.get_sparse_core_info()
    nw = info.num_cores * info.num_subcores
    per_w = n_edges // nw
    assert n_edges % nw == 0
    n_full = per_w // _CH
    rem = per_w - n_full * _CH
    assert n_full >= 6
    pairs = (n_full - 4) // 2 if n_full % 2 == 0 else (n_full - 5) // 2
    mesh = plsc.VectorSubcoreMesh(core_axis_name="c", subcore_axis_name="s")

    scratch = [
        pltpu.VMEM((2, _CH), jnp.int32), pltpu.VMEM((2, _CH), jnp.int32),
        pltpu.VMEM((2, _CH, feat), F32), pltpu.VMEM((2, _CH, feat), F32),
    ] + [pltpu.SemaphoreType.DMA] * 12
    if rem:
        scratch += [
            pltpu.VMEM((rem,), jnp.int32), pltpu.VMEM((rem,), jnp.int32),
            pltpu.VMEM((rem, feat), F32), pltpu.VMEM((rem, feat), F32),
        ]

    @functools.partial(
        pl.kernel,
        out_type=jax.ShapeDtypeStruct((n_edges, feat), F32),
        mesh=mesh,
        scratch_types=scratch,
        compiler_params=pltpu.CompilerParams(use_tc_tiling_on_sc=False),
    )
    def gather(xb_hbm, xc_hbm, src_hbm, dst_hbm, g_hbm, *refs):
        sidx, didx, ra, rb = refs[0:4]
        semis = refs[4:6]
        semid = refs[6:8]
        semga = refs[8:10]
        semgb = refs[10:12]
        semwa = refs[12:14]
        if rem:
            sidx_t, didx_t, rows_at, rows_bt = refs[16:20]
        wid = lax.axis_index("s") * info.num_cores + lax.axis_index("c")
        base0 = wid * per_w

        def i_issue(c, s):
            pltpu.async_copy(src_hbm.at[pl.ds(base0 + c * _CH, _CH)],
                             sidx.at[s], semis[s])
            pltpu.async_copy(dst_hbm.at[pl.ds(base0 + c * _CH, _CH)],
                             didx.at[s], semid[s])

        def i_wait(s):
            pltpu.make_async_copy(src_hbm.at[pl.ds(0, _CH)], sidx.at[s],
                                  semis[s]).wait()
            pltpu.make_async_copy(dst_hbm.at[pl.ds(0, _CH)], didx.at[s],
                                  semid[s]).wait()

        def g_issue(s):
            pltpu.async_copy(xb_hbm.at[sidx.at[s]], ra.at[s], semga[s])
            pltpu.async_copy(xc_hbm.at[didx.at[s]], rb.at[s], semgb[s])

        def g_wait(s):
            pltpu.make_async_copy(g_hbm.at[pl.ds(0, _CH)], ra.at[s],
                                  semga[s]).wait()
            pltpu.make_async_copy(g_hbm.at[pl.ds(0, _CH)], rb.at[s],
                                  semgb[s]).wait()

        def add_rows(s):
            def body(r, carry):
                for c in range(feat // 16):
                    plsc.addupdate(ra.at[s, r, pl.ds(c * 16, 16)],
                                   rb[s, r, pl.ds(c * 16, 16)])
                return carry

            lax.fori_loop(0, _CH, body, 0, unroll=4)

        def w_issue(c, s):
            pltpu.async_copy(ra.at[s], g_hbm.at[pl.ds(base0 + c * _CH, _CH)],
                             semwa[s])

        def w_wait(s):
            pltpu.make_async_copy(ra.at[s], g_hbm.at[pl.ds(0, _CH)],
                                  semwa[s]).wait()

        def iteration(j, s, do_next_gather, do_write_wait, do_idx_prefetch):
            s1 = 1 - s
            if do_next_gather:
                i_wait(s1)
                if do_write_wait:
                    w_wait(s1)
                g_issue(s1)
            g_wait(s)
            add_rows(s)
            w_issue(j, s)
            if do_idx_prefetch:
                i_issue(j + 2, s)

        i_issue(0, 0)
        i_issue(1, 1)
        i_wait(0)
        g_issue(0)
        iteration(0, 0, True, False, True)
        def body(g, carry):
            j = 2 * g + 1
            iteration(j, 1, True, True, True)
            iteration(j + 1, 0, True, True, True)
            return carry

        lax.fori_loop(0, pairs, body, 0)
        for j in range(2 * pairs + 1, n_full):
            iteration(j, j % 2, j + 1 < n_full, True, j + 2 < n_full)
        w_wait((n_full - 2) % 2)
        w_wait((n_full - 1) % 2)

        if rem:
            base = base0 + n_full * _CH
            pltpu.sync_copy(src_hbm.at[pl.ds(base, rem)], sidx_t)
            pltpu.sync_copy(dst_hbm.at[pl.ds(base, rem)], didx_t)
            ca = pltpu.async_copy(xb_hbm.at[sidx_t], rows_at, semga[0])
            cb = pltpu.async_copy(xc_hbm.at[didx_t], rows_bt, semgb[0])
            ca.wait()
            cb.wait()

            def body_t(r, carry):
                for c in range(feat // 16):
                    plsc.addupdate(rows_at.at[r, pl.ds(c * 16, 16)],
                                   rows_bt[r, pl.ds(c * 16, 16)])
                return carry

            lax.fori_loop(0, rem, body_t, 0, unroll=4)
            pltpu.sync_copy(rows_at, g_hbm.at[pl.ds(base, rem)])

    return gather


@functools.lru_cache(maxsize=None)
def _make_sc_scatter(n_nodes_pad, n_edges, feat):
    info = plsc.get_sparse_core_info()
    nc, ns = info.num_cores, info.num_subcores
    nw = nc * ns
    per_w = n_edges // nw
    assert n_edges % nw == 0
    n_full = per_w // _CH
    rem = per_w - n_full * _CH
    rows_per_s = n_nodes_pad // ns
    assert n_nodes_pad % (ns * _CH) == 0
    zgroups = rows_per_s // _CH
    mesh = plsc.VectorSubcoreMesh(core_axis_name="c", subcore_axis_name="s")

    assert n_full >= 6 and (n_full - 3) % 3 == 0
    scratch = [
        pltpu.VMEM((3, _CH), jnp.int32),
        pltpu.VMEM((3, _CH, feat), F32),
        pltpu.VMEM_SHARED((n_nodes_pad, feat), F32),
    ] + [pltpu.SemaphoreType.DMA] * 9
    if rem:
        scratch += [pltpu.VMEM((rem,), jnp.int32), pltpu.VMEM((rem, feat), F32)]

    @functools.partial(
        pl.kernel,
        out_type=jax.ShapeDtypeStruct((nc, n_nodes_pad, feat), F32),
        mesh=mesh,
        scratch_types=scratch,
        compiler_params=pltpu.CompilerParams(use_tc_tiling_on_sc=False),
    )
    def scatter(e_hbm, dst_hbm, out_hbm, *refs):
        didx, rows, acc = refs[0:3]
        semli = refs[3:6]
        semlr = refs[6:9]
        semsc = refs[9:12]
        if rem:
            didx_t, rows_t = refs[12:14]
        cid = lax.axis_index("c")
        sid = lax.axis_index("s")
        wid = sid * nc + cid
        base0 = wid * per_w

        def zrow(i, carry):
            for c in range(feat // 16):
                rows[0, i, pl.ds(c * 16, 16)] = jnp.zeros((16,), F32)
            return carry

        lax.fori_loop(0, _CH, zrow, 0)
        for t in range(zgroups):
            pltpu.sync_copy(rows.at[0],
                            acc.at[pl.ds(sid * rows_per_s + t * _CH, _CH)])
        plsc.subcore_barrier()

        def l_issue(c, s):
            pltpu.async_copy(dst_hbm.at[pl.ds(base0 + c * _CH, _CH)],
                             didx.at[s], semli[s])
            pltpu.async_copy(e_hbm.at[pl.ds(base0 + c * _CH, _CH)],
                             rows.at[s], semlr[s])

        def l_wait(s):
            pltpu.make_async_copy(dst_hbm.at[pl.ds(0, _CH)], didx.at[s],
                                  semli[s]).wait()
            pltpu.make_async_copy(e_hbm.at[pl.ds(0, _CH)], rows.at[s],
                                  semlr[s]).wait()

        def a_issue(s):
            pltpu.async_copy(rows.at[s], acc.at[didx.at[s]], semsc[s],
                             add=True)

        def a_wait(s):
            pltpu.make_async_copy(rows.at[s], acc.at[didx.at[s]],
                                  semsc[s]).wait()

        def iteration(j, s, do_scatter_wait, do_prefetch):
            l_wait(s)
            a_issue(s)
            if do_scatter_wait:
                a_wait((s + 2) % 3)
            if do_prefetch:
                l_issue(j + 2, (s + 2) % 3)

        l_issue(0, 0)
        l_issue(1, 1)
        l_issue(2, 2)
        iteration(0, 0, False, False)
        def body(g, carry):
            j = 3 * g + 1
            iteration(j, 1, True, True)
            iteration(j + 1, 2, True, True)
            iteration(j + 2, 0, True, True)
            return carry

        lax.fori_loop(0, (n_full - 3) // 3, body, 0)
        j0 = n_full - 2
        iteration(j0, j0 % 3, True, False)
        iteration(j0 + 1, (j0 + 1) % 3, True, False)
        a_wait((j0 + 1) % 3)

        if rem:
            base = base0 + n_full * _CH
            pltpu.sync_copy(dst_hbm.at[pl.ds(base, rem)], didx_t)
            pltpu.sync_copy(e_hbm.at[pl.ds(base, rem)], rows_t)
            pltpu.sync_copy(rows_t, acc.at[didx_t], add=True)
        plsc.subcore_barrier()

        pltpu.sync_copy(acc.at[pl.ds(sid * rows_per_s, rows_per_s)],
                        out_hbm.at[cid, pl.ds(sid * rows_per_s, rows_per_s)])

    return scatter


def _row2(v):
    return v.reshape(1, -1)


def kernel(x, edge_index, edge_attr, params):
    n, node_f = x.shape
    e_cnt = edge_index.shape[1]
    latent = params['enc_node']['ln'][0].shape[0]
    n_pad = 10240
    bn = 2000
    be = 3200

    half = e_cnt // 2
    src1 = edge_index[0, :half]
    dst1 = edge_index[1, :half]
    src2 = edge_index[0, half:]
    dst2 = edge_index[1, half:]

    (w1n, b1n), (w2n, b2n) = params['enc_node']['mlp']
    gn, ben = params['enc_node']['ln']
    xh = _mlp_ln_call(x, w1n, _row2(b1n), w2n, _row2(b2n), _row2(gn), _row2(ben), bn)

    (w1e, b1e), (w2e, b2e) = params['enc_edge']['mlp']
    ge, bee = params['enc_edge']['ln']
    e1 = _mlp_ln_call(edge_attr[:half], w1e, _row2(b1e), w2e, _row2(b2e),
                      _row2(ge), _row2(bee), be)
    e2 = _mlp_ln_call(edge_attr[half:], w1e, _row2(b1e), w2e, _row2(b2e),
                      _row2(ge), _row2(bee), be)

    sc_gather = _make_sc_gather(n, half, latent)
    sc_scatter = _make_sc_scatter(n_pad, half, latent)

    edge_grid = (half // be,)
    node_grid = (n // bn,)

    edge_call = pl.pallas_call(
        _edge_body,
        grid=edge_grid,
        in_specs=[_rows(be, (latent,)), _rows(be, (latent,)),
                  _full((latent, latent)), _full((1, latent)),
                  _full((latent, latent)), _full((1, latent)),
                  _full((1, latent)), _full((1, latent))],
        out_specs=_rows(be, (latent,)),
        out_shape=jax.ShapeDtypeStruct((half, latent), F32),
    )

    agg_spec = pl.BlockSpec((2, bn, latent), lambda i: (0, i, 0))
    node_call = pl.pallas_call(
        _node_body,
        grid=node_grid,
        in_specs=[_rows(bn, (latent,)), agg_spec, agg_spec,
                  _full((latent, latent)), _full((latent, latent)),
                  _full((1, latent)), _full((latent, latent)), _full((1, latent)),
                  _full((1, latent)), _full((1, latent)),
                  _full((latent, latent)), _full((latent, latent))],
        out_specs=[_rows(bn, (latent,))] * 3,
        out_shape=[jax.ShapeDtypeStruct((n, latent), F32)] * 3,
    )

    node_last_call = pl.pallas_call(
        _node_last_body,
        grid=node_grid,
        in_specs=[_rows(bn, (latent,)), agg_spec, agg_spec,
                  _full((latent, latent)), _full((latent, latent)),
                  _full((1, latent)), _full((latent, latent)), _full((1, latent)),
                  _full((1, latent)), _full((1, latent))],
        out_specs=_rows(bn, (latent,)),
        out_shape=jax.ShapeDtypeStruct((n, latent), F32),
    )

    tables_call = pl.pallas_call(
        _tables_body,
        grid=node_grid,
        in_specs=[_rows(bn, (latent,)), _full((latent, latent)),
                  _full((latent, latent))],
        out_specs=[_rows(bn, (latent,))] * 2,
        out_shape=[jax.ShapeDtypeStruct((n, latent), F32)] * 2,
    )

    def edge_w(p):
        (ew1, _), _ = p['edge_mlp']
        return ew1[latent:2 * latent], ew1[2 * latent:]

    b0, c0 = edge_w(params['proc'][0])
    xb, xc = tables_call(xh, b0, c0)

    n_steps = len(params['proc'])
    for s, p in enumerate(params['proc']):
        (ew1, eb1), (ew2, eb2) = p['edge_mlp']
        elg, elb = p['edge_ln']
        (nw1, nb1), (nw2, nb2) = p['node_mlp']
        nlg, nlb = p['node_ln']
        a_m = ew1[:latent]
        p_m = nw1[:latent]
        q_m = nw1[latent:]

        g1 = sc_gather(xb, xc, src1, dst1)
        g2 = sc_gather(xb, xc, src2, dst2)
        e1 = edge_call(e1, g1, a_m, _row2(eb1), ew2, _row2(eb2),
                       _row2(elg), _row2(elb))
        e2 = edge_call(e2, g2, a_m, _row2(eb1), ew2, _row2(eb2),
                       _row2(elg), _row2(elb))
        a1 = sc_scatter(e1, dst1)
        a2 = sc_scatter(e2, dst2)
        if s + 1 < n_steps:
            bn_m, cn_m = edge_w(params['proc'][s + 1])
            xh, xb, xc = node_call(xh, a1, a2, p_m, q_m, _row2(nb1), nw2,
                                   _row2(nb2), _row2(nlg), _row2(nlb),
                                   bn_m, cn_m)
        else:
            xh = node_last_call(xh, a1, a2, p_m, q_m, _row2(nb1), nw2,
                                _row2(nb2), _row2(nlg), _row2(nlb))

    (dw1, db1), (dw2, db2) = params['dec']
    out_dim = dw2.shape[1]
    pad = 8 - out_dim
    dw2p = jnp.pad(dw2, ((0, 0), (0, pad)))
    db2p = jnp.pad(db2, ((0, pad),))
    dec = pl.pallas_call(
        _dec_body,
        grid=node_grid,
        in_specs=[_rows(bn, (latent,)), _full((latent, latent)), _full((1, latent)),
                  _full((latent, 8)), _full((1, 8))],
        out_specs=_rows(bn, (8,)),
        out_shape=jax.ShapeDtypeStruct((n, 8), F32),
    )(xh, dw1, _row2(db1), dw2p, _row2(db2p))
    return dec[:, :out_dim]
